# baseline (speedup 1.0000x reference)
import functools

import jax
import jax.numpy as jnp
from jax import lax
from jax.experimental import pallas as pl
from jax.experimental.pallas import tpu as pltpu
from jax.experimental.pallas import tpu_sc as plsc

_NEG = -1e30


def _sc_pool(idx3, emb_table, B, C, D, nw, nc):
    bpw = B // nw
    ipw = bpw * C
    kc = 128
    nchunk = ipw // kc
    nh = D // 16

    mesh = plsc.VectorSubcoreMesh(core_axis_name="c", subcore_axis_name="s")

    @functools.partial(
        pl.kernel,
        out_type=jax.ShapeDtypeStruct((B, D), jnp.float32),
        mesh=mesh,
        scratch_types=[
            pltpu.VMEM((nchunk, kc), jnp.int32),
            pltpu.VMEM((ipw, D), jnp.float32),
            pltpu.VMEM((bpw, D), jnp.float32),
            pltpu.SemaphoreType.DMA,
        ],
        compiler_params=pltpu.CompilerParams(use_tc_tiling_on_sc=False),
    )
    def pool_k(idx_hbm, table_hbm, out_hbm, idx_v, rows_v, acc_v, sem):
        wid = lax.axis_index("s") * nc + lax.axis_index("c")
        pltpu.sync_copy(idx_hbm.at[wid], idx_v)
        copies = [
            pltpu.async_copy(
                table_hbm.at[idx_v.at[j]],
                rows_v.at[pl.ds(j * kc, kc)],
                sem,
            )
            for j in range(nchunk)
        ]
        for cp in copies:
            cp.wait()

        inv = jnp.float32(1.0 / C)

        def body(r, _):
            base = r * C
            for h in range(nh):
                acc = jnp.zeros((16,), jnp.float32)
                for j in range(C):
                    acc = acc + rows_v[base + j, pl.ds(h * 16, 16)]
                acc_v[r, pl.ds(h * 16, 16)] = acc * inv
            return 0

        lax.fori_loop(0, bpw, body, 0)
        pltpu.sync_copy(acc_v, out_hbm.at[pl.ds(wid * bpw, bpw)])

    return pool_k(idx3, emb_table)


def _dot_t(b_blk, a_blk):
    return lax.dot_general(
        b_blk[0], a_blk,
        (((0,), (1,)), ((), ())),
        preferred_element_type=jnp.float32,
    )


def _softmax_stats_t(a_mat, b_mat, B, K, vt, nv):
    def k(a_ref, b_ref, c_ref, m_ref, s_ref):
        v = pl.program_id(0)

        @pl.when(v == 0)
        def _():
            m_ref[...] = jnp.full((1, B), _NEG, jnp.float32)
            s_ref[...] = jnp.zeros((1, B), jnp.float32)

        logits = _dot_t(b_ref[...], a_ref[...])
        m_old = m_ref[...]
        m_new = jnp.maximum(m_old, jnp.max(logits, axis=0, keepdims=True))
        s_ref[...] = s_ref[...] * jnp.exp(m_old - m_new) + jnp.sum(
            jnp.exp(logits - m_new), axis=0, keepdims=True)
        m_ref[...] = m_new

        @pl.when(v == nv - 1)
        def _():
            c_ref[...] = m_ref[...] + jnp.log(s_ref[...])

    return pl.pallas_call(
        k,
        grid=(nv,),
        in_specs=[
            pl.BlockSpec((B, K), lambda v: (0, 0)),
            pl.BlockSpec((1, K, vt), lambda v: (v, 0, 0)),
        ],
        out_specs=pl.BlockSpec((1, B), lambda v: (0, 0)),
        out_shape=jax.ShapeDtypeStruct((1, B), jnp.float32),
        scratch_shapes=[
            pltpu.VMEM((1, B), jnp.float32),
            pltpu.VMEM((1, B), jnp.float32),
        ],
    )(a_mat, b_mat)


def _softmax_write_t(a_mat, b_mat, c, B, K, V, vt, nv):
    def k(a_ref, b_ref, c_ref, out_ref):
        logits = _dot_t(b_ref[...], a_ref[...])
        out_ref[...] = jnp.exp(logits - c_ref[...])

    return pl.pallas_call(
        k,
        grid=(nv,),
        in_specs=[
            pl.BlockSpec((B, K), lambda v: (0, 0)),
            pl.BlockSpec((1, K, vt), lambda v: (v, 0, 0)),
            pl.BlockSpec((1, B), lambda v: (0, 0)),
        ],
        out_specs=pl.BlockSpec((vt, B), lambda v: (v, 0)),
        out_shape=jax.ShapeDtypeStruct((V, B), jnp.float32),
    )(a_mat, b_mat, c)


def kernel(inputs, emb_table, fc_w, fc_b):
    B, C = inputs.shape
    V, D = emb_table.shape
    vt = 2000
    nv = V // vt

    info = plsc.get_sparse_core_info()
    nc = info.num_cores
    nw = nc * info.num_subcores
    idx3 = inputs.astype(jnp.int32).reshape(nw, B * C // (nw * 128), 128)
    pooled = _sc_pool(idx3, emb_table, B, C, D, nw, nc)

    f32 = jnp.float32
    bf16 = jnp.bfloat16
    xh = pooled.astype(bf16)
    xl = (pooled - xh.astype(f32)).astype(bf16)
    ones = jnp.ones((B, 1), bf16)
    zeros_a = jnp.zeros((B, 5), bf16)
    a_mat = jnp.concatenate([xh, xh, xl, ones, ones, ones, zeros_a], axis=1)

    wh = fc_w.astype(bf16)
    wl = (fc_w - wh.astype(f32)).astype(bf16)
    bh = fc_b.astype(bf16)
    bm = (fc_b - bh.astype(f32)).astype(bf16)
    bl = (fc_b - bh.astype(f32) - bm.astype(f32)).astype(bf16)
    zeros_b = jnp.zeros((5, V), bf16)
    b_flat = jnp.concatenate(
        [wh, wl, wh, bh[None], bm[None], bl[None], zeros_b], axis=0)
    K = a_mat.shape[1]
    b_mat = b_flat.reshape(K, nv, vt).swapaxes(0, 1)
    c = _softmax_stats_t(a_mat, b_mat, B, K, vt, nv)
    out_t = _softmax_write_t(a_mat, b_mat, c, B, K, V, vt, nv)
    return out_t.T
